# baseline (speedup 1.0000x reference)
import functools

import jax
import jax.numpy as jnp
from jax import lax
from jax.experimental import pallas as pl
from jax.experimental.pallas import tpu as pltpu
from jax.experimental.pallas import tpu_sc as plsc

N = 10000
E = 320000
D = 128
DH = D // 2
NEG_SLOPE = 0.01

NC = 2
NS = 16
L = 16
NW = NC * NS
EP = E // NW
ES = E // NS
K = 128
NCHUNK = 158
ESP = NCHUNK * K
NPAD = 10240
NTRASH = NPAD - N
RPT = NPAD // NS
ZB = RPT // K

_sc_mesh = plsc.VectorSubcoreMesh(core_axis_name="c", subcore_axis_name="s")


def _deg_body(src_hbm, dst_hbm, odeg_hbm, ideg_hbm, dstp_hbm,
              src_v, dst_v, dstp_v, odeg_v, ideg_v):
    wid = lax.axis_index("s") * NC + lax.axis_index("c")
    pltpu.sync_copy(src_hbm.at[wid], src_v)
    pltpu.sync_copy(dst_hbm.at[wid], dst_v)

    zero16 = jnp.zeros((L,), jnp.float32)

    def zero_body(i, carry):
        odeg_v[pl.ds(i * L, L)] = zero16
        ideg_v[pl.ds(i * L, L)] = zero16
        return carry

    lax.fori_loop(0, N // L, zero_body, 0)

    ones16 = jnp.ones((L,), jnp.float32)
    trash16 = jnp.full((L,), N, jnp.int32)

    def body(i, carry):
        s16 = src_v[pl.ds(i * L, L)]
        d16 = dst_v[pl.ds(i * L, L)]
        m = s16 != d16
        plsc.addupdate_scatter(odeg_v, [s16], ones16, mask=m)
        plsc.addupdate_scatter(ideg_v, [d16], ones16, mask=m)
        dstp_v[pl.ds(i * L, L)] = jnp.where(m, d16, trash16)
        return carry

    lax.fori_loop(0, EP // L, body, 0)

    pltpu.sync_copy(odeg_v, odeg_hbm.at[wid])
    pltpu.sync_copy(ideg_v, ideg_hbm.at[wid])
    pltpu.sync_copy(dstp_v, dstp_hbm.at[wid])


_deg_call = functools.partial(
    pl.kernel,
    out_type=(
        jax.ShapeDtypeStruct((NW, N), jnp.float32),
        jax.ShapeDtypeStruct((NW, N), jnp.float32),
        jax.ShapeDtypeStruct((NW, EP), jnp.int32),
    ),
    mesh=_sc_mesh,
    scratch_types=[
        pltpu.VMEM((EP,), jnp.int32),
        pltpu.VMEM((EP,), jnp.int32),
        pltpu.VMEM((EP,), jnp.int32),
        pltpu.VMEM((N,), jnp.float32),
        pltpu.VMEM((N,), jnp.float32),
    ],
    compiler_params=pltpu.CompilerParams(needs_layout_passes=False),
)(_deg_body)


def _spmm_body(msg_hbm, srcr_hbm, dstr_hbm, out_hbm,
               sidx_v, didx_v, r0, r1, r2, r3, accum_sh,
               gs0, gs1, gs2, gs3, ss0, ss1, ss2, ss3):
    cid = lax.axis_index("c")
    sid = lax.axis_index("s")
    bufs = (r0, r1, r2, r3)
    gsems = (gs0, gs1, gs2, gs3)
    ssems = (ss0, ss1, ss2, ss3)
    mhalf = msg_hbm.at[cid]

    pltpu.sync_copy(srcr_hbm.at[sid], sidx_v)
    pltpu.sync_copy(dstr_hbm.at[sid], didx_v)

    zero16 = jnp.zeros((L,), jnp.float32)

    def zrow(i, carry):
        r = i // (DH // L)
        c = i % (DH // L)
        r0[r, pl.ds(c * L, L)] = zero16
        return carry

    lax.fori_loop(0, K * DH // L, zrow, 0)

    def zcopy(j, carry):
        pltpu.sync_copy(r0, accum_sh.at[pl.ds(sid * RPT + j * K, K)])
        return carry

    lax.fori_loop(0, ZB, zcopy, 0)
    plsc.subcore_barrier()

    pltpu.async_copy(mhalf.at[sidx_v.at[0]], r0, gs0)
    pltpu.async_copy(mhalf.at[sidx_v.at[1]], r1, gs1)

    def outer(o, carry):
        for b in range(4):
            j = o * 4 + b
            b2 = (b + 2) % 4
            pltpu.make_async_copy(mhalf.at[sidx_v.at[j]], bufs[b],
                                  gsems[b]).wait()
            pltpu.async_copy(bufs[b], accum_sh.at[didx_v.at[j]], ssems[b],
                             add=True)

            @pl.when(j >= 2)
            def _():
                pltpu.make_async_copy(bufs[b2], accum_sh.at[didx_v.at[j - 2]],
                                      ssems[b2]).wait()

            @pl.when(j + 2 < NCHUNK)
            def _():
                pltpu.async_copy(mhalf.at[sidx_v.at[j + 2]], bufs[b2],
                                 gsems[b2])
        return carry

    lax.fori_loop(0, NCHUNK // 4, outer, 0)
    for b in range(2):
        j = NCHUNK - 2 + b
        b2 = (b + 2) % 4
        pltpu.make_async_copy(mhalf.at[sidx_v.at[j]], bufs[b],
                              gsems[b]).wait()
        pltpu.async_copy(bufs[b], accum_sh.at[didx_v.at[j]], ssems[b],
                         add=True)
        pltpu.make_async_copy(bufs[b2], accum_sh.at[didx_v.at[j - 2]],
                              ssems[b2]).wait()
    pltpu.make_async_copy(r0, accum_sh.at[didx_v.at[NCHUNK - 2]], ss0).wait()
    pltpu.make_async_copy(r1, accum_sh.at[didx_v.at[NCHUNK - 1]], ss1).wait()
    plsc.subcore_barrier()

    pltpu.sync_copy(accum_sh.at[pl.ds(sid * RPT, RPT)], out_hbm.at[cid, sid])


_spmm_call = functools.partial(
    pl.kernel,
    out_type=jax.ShapeDtypeStruct((NC, NS, RPT, DH), jnp.float32),
    mesh=_sc_mesh,
    scratch_types=[
        pltpu.VMEM((NCHUNK, K), jnp.int32),
        pltpu.VMEM((NCHUNK, K), jnp.int32),
        pltpu.VMEM((K, DH), jnp.float32),
        pltpu.VMEM((K, DH), jnp.float32),
        pltpu.VMEM((K, DH), jnp.float32),
        pltpu.VMEM((K, DH), jnp.float32),
        pltpu.VMEM_SHARED((NPAD, DH), jnp.float32),
        pltpu.SemaphoreType.DMA,
        pltpu.SemaphoreType.DMA,
        pltpu.SemaphoreType.DMA,
        pltpu.SemaphoreType.DMA,
        pltpu.SemaphoreType.DMA,
        pltpu.SemaphoreType.DMA,
        pltpu.SemaphoreType.DMA,
        pltpu.SemaphoreType.DMA,
    ],
    compiler_params=pltpu.CompilerParams(use_tc_tiling_on_sc=False),
)(_spmm_body)


def _prep_body(odeg_ref, ideg_ref, nsrc_ref, ndst_ref):
    od = jnp.sum(odeg_ref[...], axis=0, keepdims=True) + 1.0
    idg = jnp.sum(ideg_ref[...], axis=0, keepdims=True) + 1.0
    nsrc_ref[...] = lax.rsqrt(od)
    ndst_ref[...] = lax.rsqrt(idg)


def _prep(odeg_p, ideg_p):
    return pl.pallas_call(
        _prep_body,
        out_shape=(
            jax.ShapeDtypeStruct((1, N), jnp.float32),
            jax.ShapeDtypeStruct((1, N), jnp.float32),
        ),
    )(odeg_p, ideg_p)


BLK = 1000
GRID = N // BLK


def _scale_body(x_ref, s_ref, o_ref):
    y = x_ref[...] * s_ref[...]
    o_ref[0] = y[:, :DH]
    o_ref[1] = y[:, DH:]


def _scale(x, s_col):
    return pl.pallas_call(
        _scale_body,
        grid=(GRID,),
        in_specs=[
            pl.BlockSpec((BLK, D), lambda i: (i, 0)),
            pl.BlockSpec((BLK, 1), lambda i: (i, 0)),
        ],
        out_specs=pl.BlockSpec((2, BLK, DH), lambda i: (0, i, 0)),
        out_shape=jax.ShapeDtypeStruct((2, N, DH), jnp.float32),
    )(x, s_col)


def _layer_body(split_out, p0_ref, p1_ref, m0_ref, m1_ref, ndst_ref, s_ref,
                w_ref, b_ref, o_ref):
    a0 = p0_ref[0] + m0_ref[0]
    a1 = p1_ref[0] + m1_ref[0]
    t = jnp.concatenate([a0, a1], axis=1) * ndst_ref[...]
    h = jnp.dot(t, w_ref[...], preferred_element_type=jnp.float32) + b_ref[...]
    y = jnp.where(h >= 0.0, h, h * NEG_SLOPE)
    y = y * s_ref[...]
    if split_out:
        o_ref[0] = y[:, :DH]
        o_ref[1] = y[:, DH:]
    else:
        o_ref[...] = y


def _layer(partials, msgs, ndst_col, s_col, w, b_row, split_out):
    if split_out:
        out_spec = pl.BlockSpec((2, BLK, DH), lambda i: (0, i, 0))
        out_shape = jax.ShapeDtypeStruct((2, N, DH), jnp.float32)
    else:
        out_spec = pl.BlockSpec((BLK, D), lambda i: (i, 0))
        out_shape = jax.ShapeDtypeStruct((N, D), jnp.float32)
    return pl.pallas_call(
        functools.partial(_layer_body, split_out),
        grid=(GRID,),
        in_specs=[
            pl.BlockSpec((1, BLK, DH), lambda i: (0, i, 0)),
            pl.BlockSpec((1, BLK, DH), lambda i: (1, i, 0)),
            pl.BlockSpec((1, BLK, DH), lambda i: (0, i, 0)),
            pl.BlockSpec((1, BLK, DH), lambda i: (1, i, 0)),
            pl.BlockSpec((BLK, 1), lambda i: (i, 0)),
            pl.BlockSpec((BLK, 1), lambda i: (i, 0)),
            pl.BlockSpec((D, D), lambda i: (0, 0)),
            pl.BlockSpec((1, D), lambda i: (0, 0)),
        ],
        out_specs=out_spec,
        out_shape=out_shape,
    )(partials, partials, msgs, msgs, ndst_col, s_col, w, b_row)


def kernel(in_feat, edge_index, W0, b0, W1, b1):
    src = edge_index[0]
    dst = edge_index[1]

    odeg_p, ideg_p, dstp = _deg_call(src.reshape(NW, EP), dst.reshape(NW, EP))
    nsrc_r, ndst_r = _prep(odeg_p, ideg_p)
    nsrc_c = nsrc_r.reshape(N, 1)
    ndst_c = ndst_r.reshape(N, 1)
    ones_c = jnp.ones((N, 1), jnp.float32)

    msg0 = _scale(in_feat, nsrc_c)

    npad_e = ESP - ES
    trash_tgt = N + (
        (jnp.arange(npad_e, dtype=jnp.int32)[None, :]
         + 37 * jnp.arange(NS, dtype=jnp.int32)[:, None]) % NTRASH
    )
    srcr = jnp.concatenate(
        [src.reshape(NS, ES), jnp.zeros((NS, npad_e), jnp.int32)], axis=1
    ).reshape(NS, NCHUNK, K)
    dstr = jnp.concatenate(
        [dstp.reshape(NS, ES), trash_tgt], axis=1
    ).reshape(NS, NCHUNK, K)

    part0 = _spmm_call(msg0, srcr, dstr).reshape(NC, NPAD, DH)
    msg1 = _layer(part0, msg0, ndst_c, nsrc_c, W0.astype(jnp.float32),
                  b0.reshape(1, D), split_out=True)
    part1 = _spmm_call(msg1, srcr, dstr).reshape(NC, NPAD, DH)
    out = _layer(part1, msg1, ndst_c, ones_c, W1.astype(jnp.float32),
                 b1.reshape(1, D), split_out=False)
    return out


# speedup vs baseline: 1.3868x; 1.3868x over previous
import functools

import jax
import jax.numpy as jnp
from jax import lax
from jax.experimental import pallas as pl
from jax.experimental.pallas import tpu as pltpu
from jax.experimental.pallas import tpu_sc as plsc

N = 10000
E = 320000
D = 128
DH = D // 2
NEG_SLOPE = 0.01

NC = 2
NS = 16
L = 16
NW = NC * NS
EP = E // NW
ES = E // NS
K = 80
NCHUNK = ES // K
NPAD = 10240
RPT = NPAD // NS
ZB = RPT // K
NBUF = 6
TAIL = NCHUNK - (NCHUNK // NBUF) * NBUF

_sc_mesh = plsc.VectorSubcoreMesh(core_axis_name="c", subcore_axis_name="s")


def _deg_body(src_hbm, dst_hbm, odeg_hbm, ideg_hbm, dstp_hbm,
              src_v, dst_v, dstp_v, odeg_v, ideg_v):
    wid = lax.axis_index("s") * NC + lax.axis_index("c")
    pltpu.sync_copy(src_hbm.at[wid], src_v)
    pltpu.sync_copy(dst_hbm.at[wid], dst_v)

    zero16 = jnp.zeros((L,), jnp.float32)

    def zero_body(i, carry):
        odeg_v[pl.ds(i * L, L)] = zero16
        ideg_v[pl.ds(i * L, L)] = zero16
        return carry

    lax.fori_loop(0, N // L, zero_body, 0)

    ones16 = jnp.ones((L,), jnp.float32)
    trash16 = jnp.full((L,), N, jnp.int32)

    def body(i, carry):
        s16 = src_v[pl.ds(i * L, L)]
        d16 = dst_v[pl.ds(i * L, L)]
        m = s16 != d16
        plsc.addupdate_scatter(odeg_v, [s16], ones16, mask=m)
        plsc.addupdate_scatter(ideg_v, [d16], ones16, mask=m)
        dstp_v[pl.ds(i * L, L)] = jnp.where(m, d16, trash16)
        return carry

    lax.fori_loop(0, EP // L, body, 0)

    pltpu.sync_copy(odeg_v, odeg_hbm.at[wid])
    pltpu.sync_copy(ideg_v, ideg_hbm.at[wid])
    pltpu.sync_copy(dstp_v, dstp_hbm.at[wid])


_deg_call = functools.partial(
    pl.kernel,
    out_type=(
        jax.ShapeDtypeStruct((NW, N), jnp.float32),
        jax.ShapeDtypeStruct((NW, N), jnp.float32),
        jax.ShapeDtypeStruct((NW, EP), jnp.int32),
    ),
    mesh=_sc_mesh,
    scratch_types=[
        pltpu.VMEM((EP,), jnp.int32),
        pltpu.VMEM((EP,), jnp.int32),
        pltpu.VMEM((EP,), jnp.int32),
        pltpu.VMEM((N,), jnp.float32),
        pltpu.VMEM((N,), jnp.float32),
    ],
    compiler_params=pltpu.CompilerParams(needs_layout_passes=False),
)(_deg_body)


def _spmm_body(msg_hbm, srcr_hbm, dstr_hbm, out_hbm,
               sidx_v, didx_v, r0, r1, r2, r3, r4, r5, accum_sh,
               gs0, gs1, gs2, gs3, gs4, gs5, ss0, ss1, ss2, ss3, ss4, ss5):
    cid = lax.axis_index("c")
    sid = lax.axis_index("s")
    bufs = (r0, r1, r2, r3, r4, r5)
    gsems = (gs0, gs1, gs2, gs3, gs4, gs5)
    ssems = (ss0, ss1, ss2, ss3, ss4, ss5)
    mhalf = msg_hbm.at[cid]

    pltpu.sync_copy(srcr_hbm.at[sid], sidx_v)
    pltpu.sync_copy(dstr_hbm.at[sid], didx_v)

    zero16 = jnp.zeros((L,), jnp.float32)

    def zrow(i, carry):
        r = i // (DH // L)
        c = i % (DH // L)
        r0[r, pl.ds(c * L, L)] = zero16
        return carry

    lax.fori_loop(0, K * DH // L, zrow, 0)

    def zcopy(j, carry):
        pltpu.sync_copy(r0, accum_sh.at[pl.ds(sid * RPT + j * K, K)])
        return carry

    lax.fori_loop(0, ZB, zcopy, 0)
    plsc.subcore_barrier()

    H = NBUF // 2
    for b in range(H):
        pltpu.async_copy(mhalf.at[sidx_v.at[b]], bufs[b], gsems[b])

    def outer(o, carry):
        for b in range(NBUF):
            j = o * NBUF + b
            b2 = (b + H) % NBUF
            pltpu.make_async_copy(mhalf.at[sidx_v.at[j]], bufs[b],
                                  gsems[b]).wait()
            pltpu.async_copy(bufs[b], accum_sh.at[didx_v.at[j]], ssems[b],
                             add=True)

            @pl.when(j >= H)
            def _():
                pltpu.make_async_copy(bufs[b2], accum_sh.at[didx_v.at[j - H]],
                                      ssems[b2]).wait()

            @pl.when(j + H < NCHUNK)
            def _():
                pltpu.async_copy(mhalf.at[sidx_v.at[j + H]], bufs[b2],
                                 gsems[b2])
        return carry

    lax.fori_loop(0, NCHUNK // NBUF, outer, 0)
    for t in range(TAIL):
        j = NCHUNK - TAIL + t
        b = j % NBUF
        b2 = (b + H) % NBUF
        pltpu.make_async_copy(mhalf.at[sidx_v.at[j]], bufs[b],
                              gsems[b]).wait()
        pltpu.async_copy(bufs[b], accum_sh.at[didx_v.at[j]], ssems[b],
                         add=True)
        pltpu.make_async_copy(bufs[b2], accum_sh.at[didx_v.at[j - H]],
                              ssems[b2]).wait()
        if j + H < NCHUNK:
            pltpu.async_copy(mhalf.at[sidx_v.at[j + H]], bufs[b2], gsems[b2])
    for t in range(H):
        j = NCHUNK - H + t
        b = j % NBUF
        pltpu.make_async_copy(bufs[b], accum_sh.at[didx_v.at[j]],
                              ssems[b]).wait()
    plsc.subcore_barrier()

    pltpu.sync_copy(accum_sh.at[pl.ds(sid * RPT, RPT)], out_hbm.at[cid, sid])


_spmm_call = functools.partial(
    pl.kernel,
    out_type=jax.ShapeDtypeStruct((NC, NS, RPT, DH), jnp.float32),
    mesh=_sc_mesh,
    scratch_types=[
        pltpu.VMEM((NCHUNK, K), jnp.int32),
        pltpu.VMEM((NCHUNK, K), jnp.int32),
    ] + [pltpu.VMEM((K, DH), jnp.float32)] * NBUF + [
        pltpu.VMEM_SHARED((NPAD, DH), jnp.float32),
    ] + [pltpu.SemaphoreType.DMA] * (2 * NBUF),
    compiler_params=pltpu.CompilerParams(use_tc_tiling_on_sc=False),
)(_spmm_body)


def _prep_body(odeg_ref, ideg_ref, nsrc_ref, ndst_ref):
    od = jnp.sum(odeg_ref[...], axis=0, keepdims=True) + 1.0
    idg = jnp.sum(ideg_ref[...], axis=0, keepdims=True) + 1.0
    nsrc_ref[...] = lax.rsqrt(od)
    ndst_ref[...] = lax.rsqrt(idg)


def _prep(odeg_p, ideg_p):
    return pl.pallas_call(
        _prep_body,
        out_shape=(
            jax.ShapeDtypeStruct((1, N), jnp.float32),
            jax.ShapeDtypeStruct((1, N), jnp.float32),
        ),
    )(odeg_p, ideg_p)


BLK = 1000
GRID = N // BLK


def _scale_body(x_ref, s_ref, o_ref):
    y = x_ref[...] * s_ref[...]
    o_ref[0] = y[:, :DH]
    o_ref[1] = y[:, DH:]


def _scale(x, s_col):
    return pl.pallas_call(
        _scale_body,
        grid=(GRID,),
        in_specs=[
            pl.BlockSpec((BLK, D), lambda i: (i, 0)),
            pl.BlockSpec((BLK, 1), lambda i: (i, 0)),
        ],
        out_specs=pl.BlockSpec((2, BLK, DH), lambda i: (0, i, 0)),
        out_shape=jax.ShapeDtypeStruct((2, N, DH), jnp.float32),
    )(x, s_col)


def _layer_body(split_out, p0_ref, p1_ref, m0_ref, m1_ref, ndst_ref, s_ref,
                w_ref, b_ref, o_ref):
    a0 = p0_ref[0] + m0_ref[0]
    a1 = p1_ref[0] + m1_ref[0]
    t = jnp.concatenate([a0, a1], axis=1) * ndst_ref[...]
    h = jnp.dot(t, w_ref[...], preferred_element_type=jnp.float32) + b_ref[...]
    y = jnp.where(h >= 0.0, h, h * NEG_SLOPE)
    y = y * s_ref[...]
    if split_out:
        o_ref[0] = y[:, :DH]
        o_ref[1] = y[:, DH:]
    else:
        o_ref[...] = y


def _layer(partials, msgs, ndst_col, s_col, w, b_row, split_out):
    if split_out:
        out_spec = pl.BlockSpec((2, BLK, DH), lambda i: (0, i, 0))
        out_shape = jax.ShapeDtypeStruct((2, N, DH), jnp.float32)
    else:
        out_spec = pl.BlockSpec((BLK, D), lambda i: (i, 0))
        out_shape = jax.ShapeDtypeStruct((N, D), jnp.float32)
    return pl.pallas_call(
        functools.partial(_layer_body, split_out),
        grid=(GRID,),
        in_specs=[
            pl.BlockSpec((1, BLK, DH), lambda i: (0, i, 0)),
            pl.BlockSpec((1, BLK, DH), lambda i: (1, i, 0)),
            pl.BlockSpec((1, BLK, DH), lambda i: (0, i, 0)),
            pl.BlockSpec((1, BLK, DH), lambda i: (1, i, 0)),
            pl.BlockSpec((BLK, 1), lambda i: (i, 0)),
            pl.BlockSpec((BLK, 1), lambda i: (i, 0)),
            pl.BlockSpec((D, D), lambda i: (0, 0)),
            pl.BlockSpec((1, D), lambda i: (0, 0)),
        ],
        out_specs=out_spec,
        out_shape=out_shape,
    )(partials, partials, msgs, msgs, ndst_col, s_col, w, b_row)


def kernel(in_feat, edge_index, W0, b0, W1, b1):
    src = edge_index[0]
    dst = edge_index[1]

    odeg_p, ideg_p, dstp = _deg_call(src.reshape(NW, EP), dst.reshape(NW, EP))
    nsrc_r, ndst_r = _prep(odeg_p, ideg_p)
    nsrc_c = nsrc_r.reshape(N, 1)
    ndst_c = ndst_r.reshape(N, 1)
    ones_c = jnp.ones((N, 1), jnp.float32)

    msg0 = _scale(in_feat, nsrc_c)

    srcr = src.reshape(NS, NCHUNK, K)
    dstr = dstp.reshape(NS, NCHUNK, K)

    part0 = _spmm_call(msg0, srcr, dstr).reshape(NC, NPAD, DH)
    msg1 = _layer(part0, msg0, ndst_c, nsrc_c, W0.astype(jnp.float32),
                  b0.reshape(1, D), split_out=True)
    part1 = _spmm_call(msg1, srcr, dstr).reshape(NC, NPAD, DH)
    out = _layer(part1, msg1, ndst_c, ones_c, W1.astype(jnp.float32),
                 b1.reshape(1, D), split_out=False)
    return out


# speedup vs baseline: 1.4553x; 1.0494x over previous
import functools

import jax
import jax.numpy as jnp
from jax import lax
from jax.experimental import pallas as pl
from jax.experimental.pallas import tpu as pltpu
from jax.experimental.pallas import tpu_sc as plsc

N = 10000
E = 320000
D = 128
DH = D // 2
NEG_SLOPE = 0.01

NC = 2
NS = 16
L = 16
NW = NC * NS
EP = E // NW
ES = E // NS
K = 80
NCHUNK = ES // K
NPAD = 10240
RPT = NPAD // NS
ZB = RPT // K
NBUF = 8
TAIL = NCHUNK - (NCHUNK // NBUF) * NBUF

_sc_mesh = plsc.VectorSubcoreMesh(core_axis_name="c", subcore_axis_name="s")


def _deg_body(src_hbm, dst_hbm, odeg_hbm, ideg_hbm, dstp_hbm,
              src_v, dst_v, dstp_v, odeg_v, ideg_v):
    wid = lax.axis_index("s") * NC + lax.axis_index("c")
    pltpu.sync_copy(src_hbm.at[wid], src_v)
    pltpu.sync_copy(dst_hbm.at[wid], dst_v)

    zero16 = jnp.zeros((L,), jnp.float32)

    def zero_body(i, carry):
        odeg_v[pl.ds(i * L, L)] = zero16
        ideg_v[pl.ds(i * L, L)] = zero16
        return carry

    lax.fori_loop(0, N // L, zero_body, 0)

    ones16 = jnp.ones((L,), jnp.float32)
    trash16 = jnp.full((L,), N, jnp.int32)

    def body(i, carry):
        s16 = src_v[pl.ds(i * L, L)]
        d16 = dst_v[pl.ds(i * L, L)]
        m = s16 != d16
        plsc.addupdate_scatter(odeg_v, [s16], ones16, mask=m)
        plsc.addupdate_scatter(ideg_v, [d16], ones16, mask=m)
        dstp_v[pl.ds(i * L, L)] = jnp.where(m, d16, trash16)
        return carry

    lax.fori_loop(0, EP // L, body, 0)

    pltpu.sync_copy(odeg_v, odeg_hbm.at[wid])
    pltpu.sync_copy(ideg_v, ideg_hbm.at[wid])
    pltpu.sync_copy(dstp_v, dstp_hbm.at[wid])


_deg_call = functools.partial(
    pl.kernel,
    out_type=(
        jax.ShapeDtypeStruct((NW, N), jnp.float32),
        jax.ShapeDtypeStruct((NW, N), jnp.float32),
        jax.ShapeDtypeStruct((NW, EP), jnp.int32),
    ),
    mesh=_sc_mesh,
    scratch_types=[
        pltpu.VMEM((EP,), jnp.int32),
        pltpu.VMEM((EP,), jnp.int32),
        pltpu.VMEM((EP,), jnp.int32),
        pltpu.VMEM((N,), jnp.float32),
        pltpu.VMEM((N,), jnp.float32),
    ],
    compiler_params=pltpu.CompilerParams(needs_layout_passes=False),
)(_deg_body)


def _spmm_body(msg_hbm, srcr_hbm, dstr_hbm, out_hbm, sidx_v, didx_v, *rest):
    cid = lax.axis_index("c")
    sid = lax.axis_index("s")
    bufs = rest[:NBUF]
    accum_sh = rest[NBUF]
    gsems = rest[NBUF + 1:2 * NBUF + 1]
    ssems = rest[2 * NBUF + 1:]
    r0 = bufs[0]
    mhalf = msg_hbm.at[cid]

    pltpu.sync_copy(srcr_hbm.at[sid], sidx_v)
    pltpu.sync_copy(dstr_hbm.at[sid], didx_v)

    zero16 = jnp.zeros((L,), jnp.float32)

    def zrow(i, carry):
        r = i // (DH // L)
        c = i % (DH // L)
        r0[r, pl.ds(c * L, L)] = zero16
        return carry

    lax.fori_loop(0, K * DH // L, zrow, 0)

    def zcopy(j, carry):
        pltpu.sync_copy(r0, accum_sh.at[pl.ds(sid * RPT + j * K, K)])
        return carry

    lax.fori_loop(0, ZB, zcopy, 0)
    plsc.subcore_barrier()

    H = NBUF // 2
    for b in range(H):
        pltpu.async_copy(mhalf.at[sidx_v.at[b]], bufs[b], gsems[b])

    def outer(o, carry):
        for b in range(NBUF):
            j = o * NBUF + b
            b2 = (b + H) % NBUF
            pltpu.make_async_copy(mhalf.at[sidx_v.at[j]], bufs[b],
                                  gsems[b]).wait()
            pltpu.async_copy(bufs[b], accum_sh.at[didx_v.at[j]], ssems[b],
                             add=True)

            @pl.when(j >= H)
            def _():
                pltpu.make_async_copy(bufs[b2], accum_sh.at[didx_v.at[j - H]],
                                      ssems[b2]).wait()

            @pl.when(j + H < NCHUNK)
            def _():
                pltpu.async_copy(mhalf.at[sidx_v.at[j + H]], bufs[b2],
                                 gsems[b2])
        return carry

    lax.fori_loop(0, NCHUNK // NBUF, outer, 0)
    for t in range(TAIL):
        j = NCHUNK - TAIL + t
        b = j % NBUF
        b2 = (b + H) % NBUF
        pltpu.make_async_copy(mhalf.at[sidx_v.at[j]], bufs[b],
                              gsems[b]).wait()
        pltpu.async_copy(bufs[b], accum_sh.at[didx_v.at[j]], ssems[b],
                         add=True)
        pltpu.make_async_copy(bufs[b2], accum_sh.at[didx_v.at[j - H]],
                              ssems[b2]).wait()
        if j + H < NCHUNK:
            pltpu.async_copy(mhalf.at[sidx_v.at[j + H]], bufs[b2], gsems[b2])
    for t in range(H):
        j = NCHUNK - H + t
        b = j % NBUF
        pltpu.make_async_copy(bufs[b], accum_sh.at[didx_v.at[j]],
                              ssems[b]).wait()
    plsc.subcore_barrier()

    pltpu.sync_copy(accum_sh.at[pl.ds(sid * RPT, RPT)], out_hbm.at[cid, sid])


_spmm_call = functools.partial(
    pl.kernel,
    out_type=jax.ShapeDtypeStruct((NC, NS, RPT, DH), jnp.float32),
    mesh=_sc_mesh,
    scratch_types=[
        pltpu.VMEM((NCHUNK, K), jnp.int32),
        pltpu.VMEM((NCHUNK, K), jnp.int32),
    ] + [pltpu.VMEM((K, DH), jnp.float32)] * NBUF + [
        pltpu.VMEM_SHARED((NPAD, DH), jnp.float32),
    ] + [pltpu.SemaphoreType.DMA] * (2 * NBUF),
    compiler_params=pltpu.CompilerParams(use_tc_tiling_on_sc=False),
)(_spmm_body)


def _prep_body(odeg_ref, ideg_ref, nsrc_ref, ndst_ref):
    od = jnp.sum(odeg_ref[...], axis=0, keepdims=True) + 1.0
    idg = jnp.sum(ideg_ref[...], axis=0, keepdims=True) + 1.0
    nsrc_ref[...] = lax.rsqrt(od)
    ndst_ref[...] = lax.rsqrt(idg)


def _prep(odeg_p, ideg_p):
    return pl.pallas_call(
        _prep_body,
        out_shape=(
            jax.ShapeDtypeStruct((1, N), jnp.float32),
            jax.ShapeDtypeStruct((1, N), jnp.float32),
        ),
    )(odeg_p, ideg_p)


BLK = 1000
GRID = N // BLK


def _scale_body(x_ref, s_ref, o_ref):
    y = x_ref[...] * s_ref[...]
    o_ref[0] = y[:, :DH]
    o_ref[1] = y[:, DH:]


def _scale(x, s_col):
    return pl.pallas_call(
        _scale_body,
        grid=(GRID,),
        in_specs=[
            pl.BlockSpec((BLK, D), lambda i: (i, 0)),
            pl.BlockSpec((BLK, 1), lambda i: (i, 0)),
        ],
        out_specs=pl.BlockSpec((2, BLK, DH), lambda i: (0, i, 0)),
        out_shape=jax.ShapeDtypeStruct((2, N, DH), jnp.float32),
    )(x, s_col)


def _layer_body(split_out, p0_ref, p1_ref, m0_ref, m1_ref, ndst_ref, s_ref,
                w_ref, b_ref, o_ref):
    a0 = p0_ref[0] + m0_ref[0]
    a1 = p1_ref[0] + m1_ref[0]
    t = jnp.concatenate([a0, a1], axis=1) * ndst_ref[...]
    h = jnp.dot(t, w_ref[...], preferred_element_type=jnp.float32) + b_ref[...]
    y = jnp.where(h >= 0.0, h, h * NEG_SLOPE)
    y = y * s_ref[...]
    if split_out:
        o_ref[0] = y[:, :DH]
        o_ref[1] = y[:, DH:]
    else:
        o_ref[...] = y


def _layer(partials, msgs, ndst_col, s_col, w, b_row, split_out):
    if split_out:
        out_spec = pl.BlockSpec((2, BLK, DH), lambda i: (0, i, 0))
        out_shape = jax.ShapeDtypeStruct((2, N, DH), jnp.float32)
    else:
        out_spec = pl.BlockSpec((BLK, D), lambda i: (i, 0))
        out_shape = jax.ShapeDtypeStruct((N, D), jnp.float32)
    return pl.pallas_call(
        functools.partial(_layer_body, split_out),
        grid=(GRID,),
        in_specs=[
            pl.BlockSpec((1, BLK, DH), lambda i: (0, i, 0)),
            pl.BlockSpec((1, BLK, DH), lambda i: (1, i, 0)),
            pl.BlockSpec((1, BLK, DH), lambda i: (0, i, 0)),
            pl.BlockSpec((1, BLK, DH), lambda i: (1, i, 0)),
            pl.BlockSpec((BLK, 1), lambda i: (i, 0)),
            pl.BlockSpec((BLK, 1), lambda i: (i, 0)),
            pl.BlockSpec((D, D), lambda i: (0, 0)),
            pl.BlockSpec((1, D), lambda i: (0, 0)),
        ],
        out_specs=out_spec,
        out_shape=out_shape,
    )(partials, partials, msgs, msgs, ndst_col, s_col, w, b_row)


def kernel(in_feat, edge_index, W0, b0, W1, b1):
    src = edge_index[0]
    dst = edge_index[1]

    odeg_p, ideg_p, dstp = _deg_call(src.reshape(NW, EP), dst.reshape(NW, EP))
    nsrc_r, ndst_r = _prep(odeg_p, ideg_p)
    nsrc_c = nsrc_r.reshape(N, 1)
    ndst_c = ndst_r.reshape(N, 1)
    ones_c = jnp.ones((N, 1), jnp.float32)

    msg0 = _scale(in_feat, nsrc_c)

    srcr = src.reshape(NS, NCHUNK, K)
    dstr = dstp.reshape(NS, NCHUNK, K)

    part0 = _spmm_call(msg0, srcr, dstr).reshape(NC, NPAD, DH)
    msg1 = _layer(part0, msg0, ndst_c, nsrc_c, W0.astype(jnp.float32),
                  b0.reshape(1, D), split_out=True)
    part1 = _spmm_call(msg1, srcr, dstr).reshape(NC, NPAD, DH)
    out = _layer(part1, msg1, ndst_c, ones_c, W1.astype(jnp.float32),
                 b1.reshape(1, D), split_out=False)
    return out


# speedup vs baseline: 1.4709x; 1.0107x over previous
import functools

import jax
import jax.numpy as jnp
from jax import lax
from jax.experimental import pallas as pl
from jax.experimental.pallas import tpu as pltpu
from jax.experimental.pallas import tpu_sc as plsc

N = 10000
E = 320000
D = 128
DH = D // 2
NEG_SLOPE = 0.01

NC = 2
NS = 16
L = 16
NW = NC * NS
EP = E // NW
ES = E // NS
K = 80
NCHUNK = ES // K
NPAD = 10240
RPT = NPAD // NS
ZB = RPT // K
NBUF = 8
TAIL = NCHUNK - (NCHUNK // NBUF) * NBUF

_sc_mesh = plsc.VectorSubcoreMesh(core_axis_name="c", subcore_axis_name="s")


def _deg_body(src_hbm, dst_hbm, odeg_hbm, ideg_hbm, dstp_hbm,
              src_v, dst_v, dstp_v, odeg_v, ideg_v):
    wid = lax.axis_index("s") * NC + lax.axis_index("c")
    pltpu.sync_copy(src_hbm.at[wid], src_v)
    pltpu.sync_copy(dst_hbm.at[wid], dst_v)

    zero16 = jnp.zeros((L,), jnp.float32)

    def zero_body(i, carry):
        odeg_v[pl.ds(i * L, L)] = zero16
        ideg_v[pl.ds(i * L, L)] = zero16
        return carry

    lax.fori_loop(0, N // L, zero_body, 0)

    ones16 = jnp.ones((L,), jnp.float32)
    trash16 = jnp.full((L,), N, jnp.int32)

    def body(i, carry):
        s16 = src_v[pl.ds(i * L, L)]
        d16 = dst_v[pl.ds(i * L, L)]
        m = s16 != d16
        plsc.addupdate_scatter(odeg_v, [s16], ones16, mask=m)
        plsc.addupdate_scatter(ideg_v, [d16], ones16, mask=m)
        dstp_v[pl.ds(i * L, L)] = jnp.where(m, d16, trash16)
        return carry

    lax.fori_loop(0, EP // L, body, 0)

    pltpu.sync_copy(odeg_v, odeg_hbm.at[wid])
    pltpu.sync_copy(ideg_v, ideg_hbm.at[wid])
    pltpu.sync_copy(dstp_v, dstp_hbm.at[wid])


_deg_call = functools.partial(
    pl.kernel,
    out_type=(
        jax.ShapeDtypeStruct((NW, N), jnp.float32),
        jax.ShapeDtypeStruct((NW, N), jnp.float32),
        jax.ShapeDtypeStruct((NW, EP), jnp.int32),
    ),
    mesh=_sc_mesh,
    scratch_types=[
        pltpu.VMEM((EP,), jnp.int32),
        pltpu.VMEM((EP,), jnp.int32),
        pltpu.VMEM((EP,), jnp.int32),
        pltpu.VMEM((N,), jnp.float32),
        pltpu.VMEM((N,), jnp.float32),
    ],
    compiler_params=pltpu.CompilerParams(needs_layout_passes=False),
)(_deg_body)


def _spmm_body(msg_hbm, srcr_hbm, dstr_hbm, out_hbm, sidx_v, didx_v, *rest):
    cid = lax.axis_index("c")
    sid = lax.axis_index("s")
    bufs = rest[:NBUF]
    accum_sh = rest[NBUF]
    gsems = rest[NBUF + 1:2 * NBUF + 1]
    ssems = rest[2 * NBUF + 1:]
    r0 = bufs[0]
    mhalf = msg_hbm.at[cid]

    pltpu.sync_copy(srcr_hbm.at[sid], sidx_v)
    pltpu.sync_copy(dstr_hbm.at[sid], didx_v)

    @pl.when(sid < NS - 1)
    def _():
        pltpu.sync_copy(mhalf.at[pl.ds(sid * RPT, RPT)],
                        accum_sh.at[pl.ds(sid * RPT, RPT)])

    @pl.when(sid == NS - 1)
    def _():
        pltpu.sync_copy(mhalf.at[pl.ds((NS - 1) * RPT, N - (NS - 1) * RPT)],
                        accum_sh.at[pl.ds((NS - 1) * RPT, N - (NS - 1) * RPT)])

    plsc.subcore_barrier()

    H = NBUF // 2
    for b in range(H):
        pltpu.async_copy(mhalf.at[sidx_v.at[b]], bufs[b], gsems[b])

    def outer(o, carry):
        for b in range(NBUF):
            j = o * NBUF + b
            b2 = (b + H) % NBUF
            pltpu.make_async_copy(mhalf.at[sidx_v.at[j]], bufs[b],
                                  gsems[b]).wait()
            pltpu.async_copy(bufs[b], accum_sh.at[didx_v.at[j]], ssems[b],
                             add=True)

            @pl.when(j >= H)
            def _():
                pltpu.make_async_copy(bufs[b2], accum_sh.at[didx_v.at[j - H]],
                                      ssems[b2]).wait()

            @pl.when(j + H < NCHUNK)
            def _():
                pltpu.async_copy(mhalf.at[sidx_v.at[j + H]], bufs[b2],
                                 gsems[b2])
        return carry

    lax.fori_loop(0, NCHUNK // NBUF, outer, 0)
    for t in range(TAIL):
        j = NCHUNK - TAIL + t
        b = j % NBUF
        b2 = (b + H) % NBUF
        pltpu.make_async_copy(mhalf.at[sidx_v.at[j]], bufs[b],
                              gsems[b]).wait()
        pltpu.async_copy(bufs[b], accum_sh.at[didx_v.at[j]], ssems[b],
                         add=True)
        pltpu.make_async_copy(bufs[b2], accum_sh.at[didx_v.at[j - H]],
                              ssems[b2]).wait()
        if j + H < NCHUNK:
            pltpu.async_copy(mhalf.at[sidx_v.at[j + H]], bufs[b2], gsems[b2])
    for t in range(H):
        j = NCHUNK - H + t
        b = j % NBUF
        pltpu.make_async_copy(bufs[b], accum_sh.at[didx_v.at[j]],
                              ssems[b]).wait()
    plsc.subcore_barrier()

    pltpu.sync_copy(accum_sh.at[pl.ds(sid * RPT, RPT)], out_hbm.at[cid, sid])


_spmm_call = functools.partial(
    pl.kernel,
    out_type=jax.ShapeDtypeStruct((NC, NS, RPT, DH), jnp.float32),
    mesh=_sc_mesh,
    scratch_types=[
        pltpu.VMEM((NCHUNK, K), jnp.int32),
        pltpu.VMEM((NCHUNK, K), jnp.int32),
    ] + [pltpu.VMEM((K, DH), jnp.float32)] * NBUF + [
        pltpu.VMEM_SHARED((NPAD, DH), jnp.float32),
    ] + [pltpu.SemaphoreType.DMA] * (2 * NBUF),
    compiler_params=pltpu.CompilerParams(use_tc_tiling_on_sc=False),
)(_spmm_body)


def _prep_body(odeg_ref, ideg_ref, nsrc_ref, ndst_ref):
    od = jnp.sum(odeg_ref[...], axis=0, keepdims=True) + 1.0
    idg = jnp.sum(ideg_ref[...], axis=0, keepdims=True) + 1.0
    nsrc_ref[...] = lax.rsqrt(od)
    ndst_ref[...] = lax.rsqrt(idg)


def _prep(odeg_p, ideg_p):
    return pl.pallas_call(
        _prep_body,
        out_shape=(
            jax.ShapeDtypeStruct((1, N), jnp.float32),
            jax.ShapeDtypeStruct((1, N), jnp.float32),
        ),
    )(odeg_p, ideg_p)


BLK = 1000
GRID = N // BLK


def _scale_body(x_ref, s_ref, o_ref):
    y = x_ref[...] * s_ref[...]
    o_ref[0] = y[:, :DH]
    o_ref[1] = y[:, DH:]


def _scale(x, s_col):
    return pl.pallas_call(
        _scale_body,
        grid=(GRID,),
        in_specs=[
            pl.BlockSpec((BLK, D), lambda i: (i, 0)),
            pl.BlockSpec((BLK, 1), lambda i: (i, 0)),
        ],
        out_specs=pl.BlockSpec((2, BLK, DH), lambda i: (0, i, 0)),
        out_shape=jax.ShapeDtypeStruct((2, N, DH), jnp.float32),
    )(x, s_col)


def _layer_body(split_out, p0_ref, p1_ref, ndst_ref, s_ref,
                w_ref, b_ref, o_ref):
    t = jnp.concatenate([p0_ref[0], p1_ref[0]], axis=1) * ndst_ref[...]
    h = jnp.dot(t, w_ref[...], preferred_element_type=jnp.float32) + b_ref[...]
    y = jnp.where(h >= 0.0, h, h * NEG_SLOPE)
    y = y * s_ref[...]
    if split_out:
        o_ref[0] = y[:, :DH]
        o_ref[1] = y[:, DH:]
    else:
        o_ref[...] = y


def _layer(partials, ndst_col, s_col, w, b_row, split_out):
    if split_out:
        out_spec = pl.BlockSpec((2, BLK, DH), lambda i: (0, i, 0))
        out_shape = jax.ShapeDtypeStruct((2, N, DH), jnp.float32)
    else:
        out_spec = pl.BlockSpec((BLK, D), lambda i: (i, 0))
        out_shape = jax.ShapeDtypeStruct((N, D), jnp.float32)
    return pl.pallas_call(
        functools.partial(_layer_body, split_out),
        grid=(GRID,),
        in_specs=[
            pl.BlockSpec((1, BLK, DH), lambda i: (0, i, 0)),
            pl.BlockSpec((1, BLK, DH), lambda i: (1, i, 0)),
            pl.BlockSpec((BLK, 1), lambda i: (i, 0)),
            pl.BlockSpec((BLK, 1), lambda i: (i, 0)),
            pl.BlockSpec((D, D), lambda i: (0, 0)),
            pl.BlockSpec((1, D), lambda i: (0, 0)),
        ],
        out_specs=out_spec,
        out_shape=out_shape,
    )(partials, partials, ndst_col, s_col, w, b_row)


def kernel(in_feat, edge_index, W0, b0, W1, b1):
    src = edge_index[0]
    dst = edge_index[1]

    odeg_p, ideg_p, dstp = _deg_call(src.reshape(NW, EP), dst.reshape(NW, EP))
    nsrc_r, ndst_r = _prep(odeg_p, ideg_p)
    nsrc_c = nsrc_r.reshape(N, 1)
    ndst_c = ndst_r.reshape(N, 1)
    ones_c = jnp.ones((N, 1), jnp.float32)

    msg0 = _scale(in_feat, nsrc_c)

    srcr = src.reshape(NS, NCHUNK, K)
    dstr = dstp.reshape(NS, NCHUNK, K)

    part0 = _spmm_call(msg0, srcr, dstr).reshape(NC, NPAD, DH)
    msg1 = _layer(part0, ndst_c, nsrc_c, W0.astype(jnp.float32),
                  b0.reshape(1, D), split_out=True)
    part1 = _spmm_call(msg1, srcr, dstr).reshape(NC, NPAD, DH)
    out = _layer(part1, ndst_c, ones_c, W1.astype(jnp.float32),
                 b1.reshape(1, D), split_out=False)
    return out
